# Initial kernel scaffold; baseline (speedup 1.0000x reference)
import jax
import jax.numpy as jnp
from jax.experimental import pallas as pl


def kernel(x, edge_index, W1, b1, W2, b2):
    raise NotImplementedError("write your pallas kernel here")


# speedup vs baseline: 1.0862x; 1.0862x over previous
import functools

import jax
import jax.numpy as jnp
from jax import lax
from jax.experimental import pallas as pl
from jax.experimental.pallas import tpu as pltpu
from jax.experimental.pallas import tpu_sc as plsc

FEAT = 128
N_NODES = 10000
N_EDGES = 320000
NW = 32
E_PER_W = N_EDGES // NW
CHUNK = 80
N_CHUNKS = E_PER_W // CHUNK
GROUPS = CHUNK // 16
LANES = 16


def _precompute_tables(x, w1a, w1b, b1r):
    rows = 2000
    grid = x.shape[0] // rows

    def body(x_ref, wa_ref, wb_ref, b1_ref, a_ref, b_ref):
        xb = x_ref[...]
        a_ref[...] = (
            jnp.dot(xb, wa_ref[...], preferred_element_type=jnp.float32)
            + b1_ref[...]
        )
        b_ref[...] = jnp.dot(xb, wb_ref[...], preferred_element_type=jnp.float32)

    return pl.pallas_call(
        body,
        grid=(grid,),
        in_specs=[
            pl.BlockSpec((rows, FEAT), lambda i: (i, 0)),
            pl.BlockSpec((FEAT, FEAT), lambda i: (0, 0)),
            pl.BlockSpec((FEAT, FEAT), lambda i: (0, 0)),
            pl.BlockSpec((1, FEAT), lambda i: (0, 0)),
        ],
        out_specs=[
            pl.BlockSpec((rows, FEAT), lambda i: (i, 0)),
            pl.BlockSpec((rows, FEAT), lambda i: (i, 0)),
        ],
        out_shape=[
            jax.ShapeDtypeStruct((x.shape[0], FEAT), jnp.float32),
            jax.ShapeDtypeStruct((x.shape[0], FEAT), jnp.float32),
        ],
    )(x, w1a, w1b, b1r)


def _edge_scores(a_tab, b_tab, src, dst, w2bc, b2v):
    mesh = plsc.VectorSubcoreMesh(core_axis_name="c", subcore_axis_name="s")

    @functools.partial(
        pl.kernel,
        mesh=mesh,
        out_type=jax.ShapeDtypeStruct((N_EDGES,), jnp.float32),
        scratch_types=[
            pltpu.VMEM((CHUNK,), jnp.int32),
            pltpu.VMEM((CHUNK,), jnp.int32),
            pltpu.VMEM((CHUNK, FEAT), jnp.float32),
            pltpu.VMEM((CHUNK, FEAT), jnp.float32),
            pltpu.VMEM((CHUNK,), jnp.float32),
            pltpu.VMEM((FEAT, LANES), jnp.float32),
            pltpu.VMEM((LANES,), jnp.float32),
            pltpu.SemaphoreType.DMA,
            pltpu.SemaphoreType.DMA,
        ],
        compiler_params=pltpu.CompilerParams(needs_layout_passes=False),
    )
    def k(a_hbm, b_hbm, src_hbm, dst_hbm, w2_hbm, b2_hbm, out_hbm,
          sidx, didx, arows, brows, outv, w2v, b2vv, sem_a, sem_b):
        wid = lax.axis_index("s") * 2 + lax.axis_index("c")
        base0 = wid * E_PER_W
        pltpu.sync_copy(w2_hbm, w2v)
        pltpu.sync_copy(b2_hbm, b2vv)
        b2vec = b2vv[...]

        def chunk_body(ci, carry):
            base = base0 + ci * CHUNK
            pltpu.sync_copy(src_hbm.at[pl.ds(base, CHUNK)], sidx)
            pltpu.sync_copy(dst_hbm.at[pl.ds(base, CHUNK)], didx)
            ca = pltpu.async_copy(a_hbm.at[sidx], arows, sem_a)
            cb = pltpu.async_copy(b_hbm.at[didx], brows, sem_b)
            ca.wait()
            cb.wait()
            for g in range(GROUPS):
                grows = lax.broadcasted_iota(jnp.int32, (LANES,), 0) + (g * LANES)

                def f_body(f, acc):
                    cols = jnp.full((LANES,), f, jnp.int32)
                    ga = plsc.load_gather(arows, [grows, cols])
                    gb = plsc.load_gather(brows, [grows, cols])
                    w2f = w2v[f, :]
                    return acc + jnp.maximum(ga + gb, 0.0) * w2f

                logits = lax.fori_loop(
                    0, FEAT, f_body, jnp.zeros((LANES,), jnp.float32)
                ) + b2vec
                outv[pl.ds(g * LANES, LANES)] = 1.0 / (1.0 + jnp.exp(-logits))
            pltpu.sync_copy(outv, out_hbm.at[pl.ds(base, CHUNK)])
            return carry

        lax.fori_loop(0, N_CHUNKS, chunk_body, 0)

    return k(a_tab, b_tab, src, dst, w2bc, b2v)


def kernel(x, edge_index, W1, b1, W2, b2):
    w1a = W1[:FEAT]
    w1b = W1[FEAT:]
    b1r = b1.reshape(1, FEAT)
    a_tab, b_tab = _precompute_tables(x, w1a, w1b, b1r)
    src = edge_index[0]
    dst = edge_index[1]
    w2bc = jnp.broadcast_to(W2.reshape(FEAT, 1), (FEAT, LANES))
    b2v = jnp.broadcast_to(b2, (LANES,))
    return _edge_scores(a_tab, b_tab, src, dst, w2bc, b2v)


# speedup vs baseline: 1.2904x; 1.1880x over previous
import functools

import jax
import jax.numpy as jnp
from jax import lax
from jax.experimental import pallas as pl
from jax.experimental.pallas import tpu as pltpu
from jax.experimental.pallas import tpu_sc as plsc

FEAT = 128
N_NODES = 10000
N_EDGES = 320000
NW = 32
E_PER_W = N_EDGES // NW
CHUNK = 80
N_CHUNKS = E_PER_W // CHUNK
N_PAIRS = (N_CHUNKS - 1) // 2
LANES = 16
KSLICE = FEAT // LANES


def _precompute_tables(x, w1a, w1b, b1r):
    rows = 2000
    grid = x.shape[0] // rows

    def body(x_ref, wa_ref, wb_ref, b1_ref, a_ref, b_ref):
        xb = x_ref[...]
        a_ref[...] = (
            jnp.dot(xb, wa_ref[...], preferred_element_type=jnp.float32)
            + b1_ref[...]
        )
        b_ref[...] = jnp.dot(xb, wb_ref[...], preferred_element_type=jnp.float32)

    return pl.pallas_call(
        body,
        grid=(grid,),
        in_specs=[
            pl.BlockSpec((rows, FEAT), lambda i: (i, 0)),
            pl.BlockSpec((FEAT, FEAT), lambda i: (0, 0)),
            pl.BlockSpec((FEAT, FEAT), lambda i: (0, 0)),
            pl.BlockSpec((1, FEAT), lambda i: (0, 0)),
        ],
        out_specs=[
            pl.BlockSpec((rows, FEAT), lambda i: (i, 0)),
            pl.BlockSpec((rows, FEAT), lambda i: (i, 0)),
        ],
        out_shape=[
            jax.ShapeDtypeStruct((x.shape[0], FEAT), jnp.float32),
            jax.ShapeDtypeStruct((x.shape[0], FEAT), jnp.float32),
        ],
    )(x, w1a, w1b, b1r)


def _edge_scores(a_tab, b_tab, src, dst, w2r, b2v):
    mesh = plsc.VectorSubcoreMesh(core_axis_name="c", subcore_axis_name="s")

    @functools.partial(
        pl.kernel,
        mesh=mesh,
        out_type=jax.ShapeDtypeStruct((N_EDGES,), jnp.float32),
        scratch_types=[
            pltpu.VMEM((E_PER_W,), jnp.int32),
            pltpu.VMEM((E_PER_W,), jnp.int32),
            pltpu.VMEM((2, CHUNK, FEAT), jnp.float32),
            pltpu.VMEM((2, CHUNK, FEAT), jnp.float32),
            pltpu.VMEM((2, CHUNK), jnp.float32),
            pltpu.VMEM((FEAT, LANES), jnp.float32),
            pltpu.VMEM((LANES,), jnp.float32),
            pltpu.SemaphoreType.DMA,
            (pltpu.SemaphoreType.DMA,) * 2,
            (pltpu.SemaphoreType.DMA,) * 2,
            (pltpu.SemaphoreType.DMA,) * 2,
        ],
        compiler_params=pltpu.CompilerParams(needs_layout_passes=False),
    )
    def k(a_hbm, b_hbm, src_hbm, dst_hbm, w2_hbm, b2_hbm, out_hbm,
          sidx, didx, arows, brows, outv, w2v, b2vv,
          sem_i, sems_a, sems_b, sems_o):
        wid = lax.axis_index("s") * 2 + lax.axis_index("c")
        base0 = wid * E_PER_W
        ci1 = pltpu.async_copy(src_hbm.at[pl.ds(base0, E_PER_W)], sidx, sem_i)
        ci2 = pltpu.async_copy(dst_hbm.at[pl.ds(base0, E_PER_W)], didx, sem_i)
        pltpu.sync_copy(w2_hbm, w2v)
        pltpu.sync_copy(b2_hbm, b2vv)
        ci1.wait()
        ci2.wait()
        b2vec = b2vv[...]

        def issue(c, buf):
            off = c * CHUNK
            pltpu.async_copy(
                a_hbm.at[sidx.at[pl.ds(off, CHUNK)]], arows.at[buf], sems_a[buf]
            )
            pltpu.async_copy(
                b_hbm.at[didx.at[pl.ds(off, CHUNK)]], brows.at[buf], sems_b[buf]
            )

        def wait_rows(buf):
            pltpu.make_async_copy(
                a_hbm.at[sidx.at[pl.ds(0, CHUNK)]], arows.at[buf], sems_a[buf]
            ).wait()
            pltpu.make_async_copy(
                b_hbm.at[didx.at[pl.ds(0, CHUNK)]], brows.at[buf], sems_b[buf]
            ).wait()

        def drain_out(buf):
            pltpu.make_async_copy(
                outv.at[buf], out_hbm.at[pl.ds(0, CHUNK)], sems_o[buf]
            ).wait()

        def compute(c, buf):
            ar = arows.at[buf]
            br = brows.at[buf]
            ov = outv.at[buf]

            @pl.when(c >= 2)
            def _():
                drain_out(buf)

            zero = jnp.zeros((LANES,), jnp.float32)
            for g in range(CHUNK // LANES):
                grows = lax.broadcasted_iota(jnp.int32, (LANES,), 0) + (g * LANES)

                @plsc.parallel_loop(0, FEAT, step=4, unroll=2,
                                    carry=(zero, zero, zero, zero))
                def f_body(f, accs):
                    outs = []
                    for t in range(4):
                        col = jnp.full((LANES,), 0, jnp.int32) + (f + t)
                        ga = plsc.load_gather(ar, [grows, col])
                        gb = plsc.load_gather(br, [grows, col])
                        outs.append(
                            jnp.maximum(ga + gb, 0.0) * w2v[f + t, :]
                        )
                    return tuple(a + o for a, o in zip(accs, outs))

                logits = (f_body[0] + f_body[1]) + (f_body[2] + f_body[3]) + b2vec
                ov[pl.ds(g * LANES, LANES)] = 1.0 / (1.0 + jnp.exp(-logits))
            pltpu.async_copy(
                ov, out_hbm.at[pl.ds(base0 + c * CHUNK, CHUNK)], sems_o[buf]
            )

        issue(0, 0)

        def pair_body(p, carry):
            c0 = 2 * p
            wait_rows(0)
            issue(c0 + 1, 1)
            compute(c0, 0)
            wait_rows(1)
            issue(c0 + 2, 0)
            compute(c0 + 1, 1)
            return carry

        lax.fori_loop(0, N_PAIRS, pair_body, 0)
        wait_rows(0)
        compute(jnp.int32(N_CHUNKS - 1), 0)
        drain_out(0)
        drain_out(1)

    return k(a_tab, b_tab, src, dst, w2r, b2v)


def kernel(x, edge_index, W1, b1, W2, b2):
    w1a = W1[:FEAT]
    w1b = W1[FEAT:]
    b1r = b1.reshape(1, FEAT)
    a_tab, b_tab = _precompute_tables(x, w1a, w1b, b1r)
    src = edge_index[0]
    dst = edge_index[1]
    w2r = jnp.broadcast_to(W2.reshape(FEAT, 1), (FEAT, LANES))
    b2v = jnp.broadcast_to(b2, (LANES,))
    return _edge_scores(a_tab, b_tab, src, dst, w2r, b2v)


# speedup vs baseline: 7.7230x; 5.9849x over previous
import functools

import jax
import jax.numpy as jnp
from jax import lax
from jax.experimental import pallas as pl
from jax.experimental.pallas import tpu as pltpu
from jax.experimental.pallas import tpu_sc as plsc

FEAT = 128
N_NODES = 10000
N_EDGES = 320000
NW = 32
E_PER_W = N_EDGES // NW
CHUNK = 80
N_CHUNKS = E_PER_W // CHUNK
N_PAIRS = (N_CHUNKS - 1) // 2
LANES = 16
KSLICE = FEAT // LANES


def _precompute_tables(x, w1a, w1b, b1r):
    rows = 2000
    grid = x.shape[0] // rows

    def body(x_ref, wa_ref, wb_ref, b1_ref, a_ref, b_ref):
        xb = x_ref[...]
        a_ref[...] = (
            jnp.dot(xb, wa_ref[...], preferred_element_type=jnp.float32)
            + b1_ref[...]
        )
        b_ref[...] = jnp.dot(xb, wb_ref[...], preferred_element_type=jnp.float32)

    return pl.pallas_call(
        body,
        grid=(grid,),
        in_specs=[
            pl.BlockSpec((rows, FEAT), lambda i: (i, 0)),
            pl.BlockSpec((FEAT, FEAT), lambda i: (0, 0)),
            pl.BlockSpec((FEAT, FEAT), lambda i: (0, 0)),
            pl.BlockSpec((1, FEAT), lambda i: (0, 0)),
        ],
        out_specs=[
            pl.BlockSpec((rows, FEAT), lambda i: (i, 0)),
            pl.BlockSpec((rows, FEAT), lambda i: (i, 0)),
        ],
        out_shape=[
            jax.ShapeDtypeStruct((x.shape[0], FEAT), jnp.float32),
            jax.ShapeDtypeStruct((x.shape[0], FEAT), jnp.float32),
        ],
    )(x, w1a, w1b, b1r)


def _edge_scores(a_tab, b_tab, src, dst, w2r, b2v):
    mesh = plsc.VectorSubcoreMesh(core_axis_name="c", subcore_axis_name="s")

    @functools.partial(
        pl.kernel,
        mesh=mesh,
        out_type=jax.ShapeDtypeStruct((N_EDGES,), jnp.float32),
        scratch_types=[
            pltpu.VMEM((E_PER_W,), jnp.int32),
            pltpu.VMEM((E_PER_W,), jnp.int32),
            pltpu.VMEM((2, CHUNK, FEAT), jnp.float32),
            pltpu.VMEM((2, CHUNK, FEAT), jnp.float32),
            pltpu.VMEM((2, CHUNK), jnp.float32),
            pltpu.VMEM((KSLICE, LANES), jnp.float32),
            pltpu.VMEM((LANES,), jnp.float32),
            pltpu.SemaphoreType.DMA,
            (pltpu.SemaphoreType.DMA,) * 2,
            (pltpu.SemaphoreType.DMA,) * 2,
            (pltpu.SemaphoreType.DMA,) * 2,
        ],
        compiler_params=pltpu.CompilerParams(needs_layout_passes=False),
    )
    def k(a_hbm, b_hbm, src_hbm, dst_hbm, w2_hbm, b2_hbm, out_hbm,
          sidx, didx, arows, brows, outv, w2v, b2vv,
          sem_i, sems_a, sems_b, sems_o):
        wid = lax.axis_index("s") * 2 + lax.axis_index("c")
        base0 = wid * E_PER_W
        ci1 = pltpu.async_copy(src_hbm.at[pl.ds(base0, E_PER_W)], sidx, sem_i)
        ci2 = pltpu.async_copy(dst_hbm.at[pl.ds(base0, E_PER_W)], didx, sem_i)
        pltpu.sync_copy(w2_hbm, w2v)
        pltpu.sync_copy(b2_hbm, b2vv)
        ci1.wait()
        ci2.wait()
        b2vec = b2vv[...]
        w2k = [w2v[kk, :] for kk in range(KSLICE)]
        lane_iota = lax.broadcasted_iota(jnp.int32, (LANES,), 0)
        last_mask = lane_iota == (LANES - 1)

        def issue(c, buf):
            off = c * CHUNK
            pltpu.async_copy(
                a_hbm.at[sidx.at[pl.ds(off, CHUNK)]], arows.at[buf], sems_a[buf]
            )
            pltpu.async_copy(
                b_hbm.at[didx.at[pl.ds(off, CHUNK)]], brows.at[buf], sems_b[buf]
            )

        def wait_rows(buf):
            pltpu.make_async_copy(
                a_hbm.at[sidx.at[pl.ds(0, CHUNK)]], arows.at[buf], sems_a[buf]
            ).wait()
            pltpu.make_async_copy(
                b_hbm.at[didx.at[pl.ds(0, CHUNK)]], brows.at[buf], sems_b[buf]
            ).wait()

        def drain_out(buf):
            pltpu.make_async_copy(
                outv.at[buf], out_hbm.at[pl.ds(0, CHUNK)], sems_o[buf]
            ).wait()

        def compute(c, buf):
            ar = arows.at[buf]
            br = brows.at[buf]
            ov = outv.at[buf]

            @pl.when(c >= 2)
            def _():
                drain_out(buf)

            @plsc.parallel_loop(0, CHUNK, unroll=4)
            def edge_body(e):
                sl0 = pl.ds(0, LANES)
                acc = jnp.maximum(ar[e, sl0] + br[e, sl0], 0.0) * w2k[0]
                for kk in range(1, KSLICE):
                    sl = pl.ds(kk * LANES, LANES)
                    acc = acc + jnp.maximum(ar[e, sl] + br[e, sl], 0.0) * w2k[kk]
                scanned = plsc.cumsum(acc)
                plsc.store_scatter(
                    ov,
                    [jnp.full((LANES,), 0, jnp.int32) + e],
                    scanned,
                    mask=last_mask,
                )

            for j in range(CHUNK // LANES):
                sl = pl.ds(j * LANES, LANES)
                ov[sl] = 1.0 / (1.0 + jnp.exp(-(ov[sl] + b2vec)))
            pltpu.async_copy(
                ov, out_hbm.at[pl.ds(base0 + c * CHUNK, CHUNK)], sems_o[buf]
            )

        issue(0, 0)

        def pair_body(p, carry):
            c0 = 2 * p
            wait_rows(0)
            issue(c0 + 1, 1)
            compute(c0, 0)
            wait_rows(1)
            issue(c0 + 2, 0)
            compute(c0 + 1, 1)
            return carry

        lax.fori_loop(0, N_PAIRS, pair_body, 0)
        wait_rows(0)
        compute(jnp.int32(N_CHUNKS - 1), 0)
        drain_out(0)
        drain_out(1)

    return k(a_tab, b_tab, src, dst, w2r, b2v)


def kernel(x, edge_index, W1, b1, W2, b2):
    w1a = W1[:FEAT]
    w1b = W1[FEAT:]
    b1r = b1.reshape(1, FEAT)
    a_tab, b_tab = _precompute_tables(x, w1a, w1b, b1r)
    src = edge_index[0]
    dst = edge_index[1]
    w2r = W2.reshape(KSLICE, LANES)
    b2v = jnp.broadcast_to(b2, (LANES,))
    return _edge_scores(a_tab, b_tab, src, dst, w2r, b2v)
